# baseline (device time: 245610 ns/iter reference)
import jax
import jax.numpy as jnp
from jax import lax
from jax.experimental import pallas as pl
from jax.experimental.pallas import tpu as pltpu

N_DEV = 4


def _all_reduce_quant(partial):
    m, n = partial.shape
    ch = m // N_DEV
    hw = n // 2

    def body(p_ref, out_ref,
             acc_r, acc_l, stage_r, stage_l, recvs_r, recvs_l,
             qmine_r, qmine_l, qrecv_r, qrecv_l, am_buf, am_recv,
             rs_send_r, rs_recv_r, rs_send_l, rs_recv_l,
             ag_send_r, ag_recv_r, ag_send_l, ag_recv_l,
             am_send_s, am_recv_s, loc_r, loc_l):
        d = lax.axis_index("i")
        right = lax.rem(d + 1, N_DEV)
        left = lax.rem(d + N_DEV - 1, N_DEV)

        barrier = pltpu.get_barrier_semaphore()
        for nbr in (left, right):
            pl.semaphore_signal(barrier, inc=1, device_id=(nbr,),
                                device_id_type=pl.DeviceIdType.MESH)
        pl.semaphore_wait(barrier, 2)

        def chunk_r(ref, idx):
            return ref.at[pl.ds(idx * ch, ch), 0:hw]

        def chunk_l(ref, idx):
            return ref.at[pl.ds(idx * ch, ch), hw:n]

        R = dict(chunk=chunk_r, recvs=recvs_r, stage=stage_r, acc=acc_r,
                 ssem=rs_send_r, rsem=rs_recv_r, loc=loc_r, nbr=right,
                 ri=lambda s: lax.rem(d - s - 1 + N_DEV, N_DEV))
        L = dict(chunk=chunk_l, recvs=recvs_l, stage=stage_l, acc=acc_l,
                 ssem=rs_send_l, rsem=rs_recv_l, loc=loc_l, nbr=left,
                 ri=lambda s: lax.rem(d + s + 1, N_DEV))

        def rs_issue(D, s):
            src = D["chunk"](p_ref, d) if s == 0 else D["acc"].at[:]
            rdma = pltpu.make_async_remote_copy(
                src_ref=src, dst_ref=D["recvs"].at[s],
                send_sem=D["ssem"].at[s], recv_sem=D["rsem"].at[s],
                device_id=(D["nbr"],), device_id_type=pl.DeviceIdType.MESH)
            rdma.start()
            cp = pltpu.make_async_copy(
                D["chunk"](p_ref, D["ri"](s)), D["stage"], D["loc"])
            cp.start()
            D["rdma"], D["cp"] = rdma, cp

        def rs_complete(D, s):
            D["cp"].wait()
            D["rdma"].wait()
            D["acc"][...] = D["recvs"][s] + D["stage"][...]
            if s < N_DEV - 2:
                rs_issue(D, s + 1)

        rs_issue(R, 0)
        rs_issue(L, 0)
        for s in range(N_DEV - 1):
            first, second = (R, L) if s % 2 == 0 else (L, R)
            rs_complete(first, s)
            rs_complete(second, s)


        m_loc = jnp.maximum(jnp.max(jnp.abs(acc_r[...])),
                            jnp.max(jnp.abs(acc_l[...])))
        am_buf[...] = jnp.full((8, 128), m_loc, jnp.float32)
        for h in range(N_DEV - 1):
            am = pltpu.make_async_remote_copy(
                src_ref=am_buf.at[:], dst_ref=am_recv.at[h],
                send_sem=am_send_s.at[h], recv_sem=am_recv_s.at[h],
                device_id=(right,), device_id_type=pl.DeviceIdType.MESH)
            am.start()
            am.wait()
            am_buf[...] = jnp.maximum(am_buf[...], am_recv[h])
        scale = jnp.max(am_buf[...]) / 448.0

        qmine_r[...] = jnp.clip(acc_r[...] / scale, -448.0, 448.0
                                ).astype(jnp.float8_e4m3fn)
        qmine_l[...] = jnp.clip(acc_l[...] / scale, -448.0, 448.0
                                ).astype(jnp.float8_e4m3fn)

        pending = []
        for t in range(N_DEV - 1):
            src_r = qmine_r.at[:] if t == 0 else qrecv_r.at[t - 1]
            src_l = qmine_l.at[:] if t == 0 else qrecv_l.at[t - 1]
            rdma_r = pltpu.make_async_remote_copy(
                src_ref=src_r, dst_ref=qrecv_r.at[t],
                send_sem=ag_send_r.at[t], recv_sem=ag_recv_r.at[t],
                device_id=(right,), device_id_type=pl.DeviceIdType.MESH)
            rdma_l = pltpu.make_async_remote_copy(
                src_ref=src_l, dst_ref=qrecv_l.at[t],
                send_sem=ag_send_l.at[t], recv_sem=ag_recv_l.at[t],
                device_id=(left,), device_id_type=pl.DeviceIdType.MESH)
            rdma_r.start()
            rdma_l.start()
            pending.append((rdma_r, rdma_l))
            if t == 0:
                q_r, q_l = qmine_r, qmine_l
                row_r = lax.rem(d + 1, N_DEV)
                row_l = lax.rem(d + N_DEV - 1, N_DEV)
            else:
                q_r, q_l = qrecv_r.at[t - 1], qrecv_l.at[t - 1]
                row_r = lax.rem(d - (t - 1) + N_DEV, N_DEV)
                row_l = lax.rem(d + (t - 1), N_DEV)
            stage_r[...] = q_r[...].astype(jnp.float32) * scale
            cp_r = pltpu.make_async_copy(stage_r, chunk_r(out_ref, row_r), loc_r)
            cp_r.start()
            stage_l[...] = q_l[...].astype(jnp.float32) * scale
            cp_l = pltpu.make_async_copy(stage_l, chunk_l(out_ref, row_l), loc_l)
            cp_l.start()
            cp_r.wait()
            cp_l.wait()
            rdma_r.wait_recv()
            rdma_l.wait_recv()
        stage_r[...] = qrecv_r[N_DEV - 2].astype(jnp.float32) * scale
        cp_r = pltpu.make_async_copy(
            stage_r, chunk_r(out_ref, lax.rem(d - (N_DEV - 2) + N_DEV, N_DEV)),
            loc_r)
        cp_r.start()
        stage_l[...] = qrecv_l[N_DEV - 2].astype(jnp.float32) * scale
        cp_l = pltpu.make_async_copy(
            stage_l, chunk_l(out_ref, lax.rem(d + N_DEV - 2, N_DEV)), loc_l)
        cp_l.start()
        cp_r.wait()
        cp_l.wait()
        for rdma_r, rdma_l in pending:
            rdma_r.wait_send()
            rdma_l.wait_send()

    nsteps = N_DEV - 1
    f8 = jnp.float8_e4m3fn
    return pl.pallas_call(
        body,
        out_shape=jax.ShapeDtypeStruct((m, n), jnp.float32),
        in_specs=[pl.BlockSpec(memory_space=pl.ANY)],
        out_specs=pl.BlockSpec(memory_space=pl.ANY),
        scratch_shapes=[
            pltpu.VMEM((ch, hw), jnp.float32),
            pltpu.VMEM((ch, hw), jnp.float32),
            pltpu.VMEM((ch, hw), jnp.float32),
            pltpu.VMEM((ch, hw), jnp.float32),
            pltpu.VMEM((nsteps, ch, hw), jnp.float32),
            pltpu.VMEM((nsteps, ch, hw), jnp.float32),
            pltpu.VMEM((ch, hw), f8),
            pltpu.VMEM((ch, hw), f8),
            pltpu.VMEM((nsteps, ch, hw), f8),
            pltpu.VMEM((nsteps, ch, hw), f8),
            pltpu.VMEM((8, 128), jnp.float32),
            pltpu.VMEM((nsteps, 8, 128), jnp.float32),
            pltpu.SemaphoreType.DMA((nsteps,)),
            pltpu.SemaphoreType.DMA((nsteps,)),
            pltpu.SemaphoreType.DMA((nsteps,)),
            pltpu.SemaphoreType.DMA((nsteps,)),
            pltpu.SemaphoreType.DMA((nsteps,)),
            pltpu.SemaphoreType.DMA((nsteps,)),
            pltpu.SemaphoreType.DMA((nsteps,)),
            pltpu.SemaphoreType.DMA((nsteps,)),
            pltpu.SemaphoreType.DMA((nsteps,)),
            pltpu.SemaphoreType.DMA((nsteps,)),
            pltpu.SemaphoreType.DMA,
            pltpu.SemaphoreType.DMA,
        ],
        compiler_params=pltpu.CompilerParams(
            collective_id=0, vmem_limit_bytes=62 * 1024 * 1024),
    )(partial)


def kernel(x, w_mat):
    partial = lax.dot(
        x, w_mat,
        precision=lax.DotAlgorithmPreset.BF16_BF16_F32_X3,
        preferred_element_type=jnp.float32,
    )
    return _all_reduce_quant(partial)
